# Initial kernel scaffold; baseline (speedup 1.0000x reference)
import jax
import jax.numpy as jnp
from jax.experimental import pallas as pl


def kernel(xyz, search_feat, template_feat, fc1_w, fc1_b, fc3_w, fc3_b, d1_w, d1_b, d2_w, d2_b, g1_w, g1_b, g2_w, g2_b, wq, wk, wv):
    raise NotImplementedError("write your pallas kernel here")


# speedup vs baseline: 13.0787x; 13.0787x over previous
import functools

import jax
import jax.numpy as jnp
from jax import lax
from jax.experimental import pallas as pl
from jax.experimental.pallas import tpu as pltpu
from jax.experimental.pallas import tpu_sc as plsc

B, N, KNN = 2, 4096, 16
DP, DM = 128, 64
TW = 3 * DM
TOT = B * N * KNN

BLK1 = 256
BLK3 = 256

NC, NS = 2, 16
NW = NC * NS
PER_W = TOT // NW
CH = 128
N_CH = PER_W // CH

_HI = jax.lax.Precision.HIGHEST


def _stage1_body(xa_ref, xat_ref, xb_ref, sf_ref, tf_ref, fc1w_ref, fc1b_ref,
                 wq_ref, wk_ref, wv_ref, d1p_ref,
                 fidx_ref, q_ref, t_ref):
    b = pl.program_id(0)

    xa = xa_ref[0]
    xat = xat_ref[0]
    na_row = jnp.sum(xat * xat, axis=0, keepdims=True)

    xb = xb_ref[0]
    nb = jnp.sum(xb * xb, axis=1, keepdims=True)

    e = lax.dot_general(xb.astype(jnp.bfloat16), xa.astype(jnp.bfloat16),
                        (((1,), (1,)), ((), ())),
                        preferred_element_type=jnp.float32)
    d = -2.0 * e + nb + na_row

    col = lax.broadcasted_iota(jnp.int32, (BLK1, N), 1)
    lane_k = lax.broadcasted_iota(jnp.int32, (BLK1, KNN), 1)
    acc = jnp.zeros((BLK1, KNN), jnp.int32)
    for j in range(KNN):
        m = jnp.min(d, axis=1, keepdims=True)
        cand = jnp.where(d == m, col, N)
        idx = jnp.min(cand, axis=1, keepdims=True)
        acc = jnp.where(lane_k == j, idx + b * N, acc)
        d = jnp.where(col == idx, jnp.float32(jnp.inf), d)
    fidx_ref[0] = acc

    fc1b = fc1b_ref[...]
    sf = lax.dot(sf_ref[0], fc1w_ref[...], precision=_HI,
                 preferred_element_type=jnp.float32) + fc1b
    tf = lax.dot(tf_ref[0], fc1w_ref[...], precision=_HI,
                 preferred_element_type=jnp.float32) + fc1b
    q_ref[0] = lax.dot(tf, wq_ref[...], precision=_HI,
                       preferred_element_type=jnp.float32)
    t_ref[0, :, 0:DM] = lax.dot(sf, wk_ref[...], precision=_HI,
                                preferred_element_type=jnp.float32)
    t_ref[0, :, DM:2 * DM] = lax.dot(sf, wv_ref[...], precision=_HI,
                                     preferred_element_type=jnp.float32)
    t_ref[0, :, 2 * DM:TW] = lax.dot(xb, d1p_ref[...], precision=_HI,
                                     preferred_element_type=jnp.float32)


def _stage1(xyz_pad, xyz_t, search_feat, template_feat, fc1_w, fc1b2, wq, wk, wv,
            d1_pad):
    grid = (B, N // BLK1)
    full = lambda shp: pl.BlockSpec(shp, lambda b, i: (0,) * len(shp))
    return pl.pallas_call(
        _stage1_body,
        grid=grid,
        in_specs=[
            pl.BlockSpec((1, N, 8), lambda b, i: (b, 0, 0)),
            pl.BlockSpec((1, 8, N), lambda b, i: (b, 0, 0)),
            pl.BlockSpec((1, BLK1, 8), lambda b, i: (b, i, 0)),
            pl.BlockSpec((1, BLK1, DP), lambda b, i: (b, i, 0)),
            pl.BlockSpec((1, BLK1, DP), lambda b, i: (b, i, 0)),
            full((DP, DM)), full((1, DM)),
            full((DM, DM)), full((DM, DM)), full((DM, DM)),
            full((8, DM)),
        ],
        out_specs=[
            pl.BlockSpec((1, BLK1, KNN), lambda b, i: (b, i, 0)),
            pl.BlockSpec((1, BLK1, DM), lambda b, i: (b, i, 0)),
            pl.BlockSpec((1, BLK1, TW), lambda b, i: (b, i, 0)),
        ],
        out_shape=[
            jax.ShapeDtypeStruct((B, N, KNN), jnp.int32),
            jax.ShapeDtypeStruct((B, N, DM), jnp.float32),
            jax.ShapeDtypeStruct((B, N, TW), jnp.float32),
        ],
    )(xyz_pad, xyz_t, xyz_pad, search_feat, template_feat, fc1_w, fc1b2,
      wq, wk, wv, d1_pad)


def _sc_gather_body(table_hbm, idx_hbm, out_hbm, idx_v, rows_v, sem):
    wid = lax.axis_index("s") * NC + lax.axis_index("c")
    base = wid * PER_W

    def chunk(i, _):
        off = base + i * CH
        pltpu.sync_copy(idx_hbm.at[pl.ds(off, CH)], idx_v)
        pltpu.async_copy(table_hbm.at[idx_v], rows_v, sem).wait()
        pltpu.sync_copy(rows_v, out_hbm.at[pl.ds(off, CH)])
        return 0

    lax.fori_loop(0, N_CH, chunk, 0)


def _sc_gather(table, fidx):
    mesh = plsc.VectorSubcoreMesh(core_axis_name="c", subcore_axis_name="s",
                                  num_cores=NC, num_subcores=NS)
    f = functools.partial(
        pl.kernel,
        out_type=jax.ShapeDtypeStruct((TOT, TW), jnp.float32),
        mesh=mesh,
        scratch_types=[
            pltpu.VMEM((CH,), jnp.int32),
            pltpu.VMEM((CH, TW), jnp.float32),
            pltpu.SemaphoreType.DMA,
        ],
        compiler_params=pltpu.CompilerParams(use_tc_tiling_on_sc=False),
    )(_sc_gather_body)
    return f(table, fidx)


def _stage3_body(g_ref, q_ref, t_ref, pre_ref,
                 d1b_ref, d2w_ref, d2b_ref, g1w_ref, g1b_ref,
                 g2w_ref, g2b_ref, fc3w_ref, fc3b_ref,
                 attn_ref, res_ref):
    R = BLK3 * KNN
    g = g_ref[...]
    kg = g[:, 0:DM]
    vg = g[:, DM:2 * DM]
    xdg = g[:, 2 * DM:TW]
    xdn = t_ref[:, 2 * DM:TW]
    q = q_ref[...]

    pe1 = jnp.maximum(
        xdn[:, None, :] - xdg.reshape(BLK3, KNN, DM) + d1b_ref[...], 0.0)
    pe = lax.dot(pe1.reshape(R, DM), d2w_ref[...], precision=_HI,
                 preferred_element_type=jnp.float32) + d2b_ref[...]
    pe3 = pe.reshape(BLK3, KNN, DM)

    t3 = q[:, None, :] - kg.reshape(BLK3, KNN, DM) + pe3
    a1 = jnp.maximum(
        lax.dot(t3.reshape(R, DM), g1w_ref[...], precision=_HI,
                preferred_element_type=jnp.float32) + g1b_ref[...], 0.0)
    al = lax.dot(a1, g2w_ref[...], precision=_HI,
                 preferred_element_type=jnp.float32) + g2b_ref[...]

    s3 = (al * 0.125).reshape(BLK3, KNN, DM)
    m = jnp.max(s3, axis=1, keepdims=True)
    ex = jnp.exp(s3 - m)
    attn3 = ex / jnp.sum(ex, axis=1, keepdims=True)
    attn_ref[...] = attn3.reshape(R, DM)

    r = jnp.sum(attn3 * (vg.reshape(BLK3, KNN, DM) + pe3), axis=1)
    res_ref[...] = (lax.dot(r, fc3w_ref[...], precision=_HI,
                            preferred_element_type=jnp.float32)
                    + fc3b_ref[...] + pre_ref[...])


def _stage3(gathered, q2, t2, pre2, d1b2, d2_w, d2b2, g1_w, g1b2, g2_w,
            g2b2, fc3_w, fc3b2):
    grid = (B * N // BLK3,)
    full = lambda shp: pl.BlockSpec(shp, lambda i: (0,) * len(shp))
    return pl.pallas_call(
        _stage3_body,
        grid=grid,
        in_specs=[
            pl.BlockSpec((BLK3 * KNN, TW), lambda i: (i, 0)),
            pl.BlockSpec((BLK3, DM), lambda i: (i, 0)),
            pl.BlockSpec((BLK3, TW), lambda i: (i, 0)),
            pl.BlockSpec((BLK3, DP), lambda i: (i, 0)),
            full((1, DM)), full((DM, DM)), full((1, DM)),
            full((DM, DM)), full((1, DM)),
            full((DM, DM)), full((1, DM)),
            full((DM, DP)), full((1, DP)),
        ],
        out_specs=[
            pl.BlockSpec((BLK3 * KNN, DM), lambda i: (i, 0)),
            pl.BlockSpec((BLK3, DP), lambda i: (i, 0)),
        ],
        out_shape=[
            jax.ShapeDtypeStruct((TOT, DM), jnp.float32),
            jax.ShapeDtypeStruct((B * N, DP), jnp.float32),
        ],
    )(gathered, q2, t2, pre2, d1b2, d2_w, d2b2, g1_w, g1b2, g2_w, g2b2,
      fc3_w, fc3b2)


def kernel(xyz, search_feat, template_feat, fc1_w, fc1_b, fc3_w, fc3_b,
           d1_w, d1_b, d2_w, d2_b, g1_w, g1_b, g2_w, g2_b, wq, wk, wv):
    xyz_pad = jnp.pad(xyz, ((0, 0), (0, 0), (0, 5)))
    d1_pad = jnp.zeros((8, DM), jnp.float32).at[:3].set(d1_w)

    xyz_t = jnp.transpose(xyz_pad, (0, 2, 1))
    fidx, q, t = _stage1(xyz_pad, xyz_t, search_feat, template_feat,
                         fc1_w, fc1_b.reshape(1, DM), wq, wk, wv, d1_pad)

    gathered = _sc_gather(t.reshape(B * N, TW), fidx.reshape(TOT))

    attn_flat, res2 = _stage3(
        gathered, q.reshape(B * N, DM), t.reshape(B * N, TW),
        search_feat.reshape(B * N, DP),
        d1_b.reshape(1, DM), d2_w, d2_b.reshape(1, DM),
        g1_w, g1_b.reshape(1, DM), g2_w, g2_b.reshape(1, DM),
        fc3_w, fc3_b.reshape(1, DP))

    return (res2.reshape(B, N, DP), attn_flat.reshape(B, N, KNN, DM))
